# baseline (speedup 1.0000x reference)
import functools

import jax
import jax.numpy as jnp
from jax import lax
from jax.experimental import pallas as pl
from jax.experimental.pallas import tpu as pltpu
from jax.experimental.pallas import tpu_sc as plsc

_NUM_CODES = 256
_EMBED_DIM = 768
_BATCH = 16384

_NC = 2
_NS = 16
_HALF = _EMBED_DIM // 2
_B_PER_W = _BATCH // _NS
_NBUF = 4
_ROWS = 16
_NBLK = _B_PER_W // _ROWS
_TA = _BATCH // 8
_TB = _EMBED_DIM // 128
_TBH = _TB // 2
_KT = _NUM_CODES // 8


@functools.partial(
    pl.kernel,
    mesh=plsc.VectorSubcoreMesh(core_axis_name="c", subcore_axis_name="s"),
    out_type=jax.ShapeDtypeStruct((_TA, _TB, 8, 128), jnp.float32),
    scratch_types=[
        pltpu.VMEM((_NUM_CODES, _HALF), jnp.float32),
        pltpu.VMEM((_B_PER_W + 16,), jnp.int32),
        pltpu.VMEM((_NBUF, _ROWS // 8, _TBH, 8, 128), jnp.float32),
        pltpu.VMEM_SHARED((_NUM_CODES, _HALF), jnp.float32),
        pltpu.SemaphoreType.DMA((_NBUF,)),
        pltpu.SemaphoreType.DMA,
    ],
    compiler_params=pltpu.CompilerParams(use_tc_tiling_on_sc=False,
                                         needs_layout_passes=False),
)
def _sc_lookup(table_hbm, idx_hbm, out_hbm, tab_v, idx_v, buf_v, tab_sp,
               wsem, sem1):
    g = lax.axis_index("s")
    h = lax.axis_index("c")
    col0 = h * _HALF
    tb0 = h * _TBH
    row_base = g * _B_PER_W
    @pl.when(g == 0)
    def _stage():
        pltpu.sync_copy(table_hbm.at[:, pl.ds(col0, _HALF)], tab_sp)
    c_idx = pltpu.async_copy(idx_hbm.at[pl.ds(row_base, _B_PER_W)],
                             idx_v.at[pl.ds(0, _B_PER_W)], sem1)
    plsc.subcore_barrier()
    pltpu.sync_copy(tab_sp, tab_v)
    c_idx.wait()

    iota16 = lax.iota(jnp.int32, 16)
    zeros16 = jnp.zeros((16,), jnp.int32)

    @pl.loop(0, _NBLK)
    def _blocks(blk):
        b = blk & (_NBUF - 1)
        r0 = blk * _ROWS
        ta0 = g * (_B_PER_W // 8) + blk * (_ROWS // 8)

        @pl.when(blk >= _NBUF)
        def _wait():
            pltpu.make_async_copy(
                buf_v.at[b],
                out_hbm.at[pl.ds(ta0, _ROWS // 8), pl.ds(tb0, _TBH)],
                wsem.at[b]).wait()

        @plsc.parallel_loop(0, _ROWS)
        def _row(r):
            idxv = idx_v[pl.ds(r0 + r, 16)]
            bcast = idxv.at[zeros16].get(mode="promise_in_bounds")
            ra = r // 8
            rs = r % 8
            for k in range(_HALF // 16):
                v = plsc.load_gather(tab_v, [bcast, iota16 + k * 16])
                dst = buf_v.at[b, ra, k // 8, rs]
                dst[pl.ds((k % 8) * 16, 16)] = v

        pltpu.async_copy(
            buf_v.at[b],
            out_hbm.at[pl.ds(ta0, _ROWS // 8), pl.ds(tb0, _TBH)],
            wsem.at[b])

    for b in range(_NBUF):
        pltpu.make_async_copy(
            buf_v.at[b],
            out_hbm.at[pl.ds(0, _ROWS // 8), pl.ds(tb0, _TBH)],
            wsem.at[b]).wait()


def kernel(anchor_embeddings, indices):
    tiled = _sc_lookup(anchor_embeddings, indices.astype(jnp.int32))
    return tiled.transpose(0, 2, 1, 3).reshape(_BATCH, _EMBED_DIM)


# speedup vs baseline: 1.0463x; 1.0463x over previous
import functools

import jax
import jax.numpy as jnp
from jax import lax
from jax.experimental import pallas as pl
from jax.experimental.pallas import tpu as pltpu
from jax.experimental.pallas import tpu_sc as plsc

_NUM_CODES = 256
_EMBED_DIM = 768
_BATCH = 16384

_NC = 2
_NS = 16
_HALF = _EMBED_DIM // 2
_B_PER_W = _BATCH // _NS
_ROWS = 32
_NBLK = _B_PER_W // _ROWS
_TA = _BATCH // 8
_TB = _EMBED_DIM // 128
_TBH = _TB // 2
_KT = _NUM_CODES // 8


@functools.partial(
    pl.kernel,
    mesh=plsc.VectorSubcoreMesh(core_axis_name="c", subcore_axis_name="s"),
    out_type=jax.ShapeDtypeStruct((_TA, _TB, 8, 128), jnp.float32),
    scratch_types=[
        pltpu.VMEM((_NUM_CODES, _HALF), jnp.float32),
        pltpu.VMEM((_B_PER_W + 16,), jnp.int32),
        pltpu.VMEM((2, _ROWS // 8, _TBH, 8, 128), jnp.float32),
        pltpu.VMEM_SHARED((_NUM_CODES, _HALF), jnp.float32),
        pltpu.SemaphoreType.DMA((2,)),
        pltpu.SemaphoreType.DMA,
    ],
    compiler_params=pltpu.CompilerParams(use_tc_tiling_on_sc=False,
                                         needs_layout_passes=False),
)
def _sc_lookup(table_hbm, idx_hbm, out_hbm, tab_v, idx_v, buf_v, tab_sp,
               wsem, sem1):
    g = lax.axis_index("s")
    h = lax.axis_index("c")
    col0 = h * _HALF
    tb0 = h * _TBH
    row_base = g * _B_PER_W
    @pl.when(g == 0)
    def _stage():
        pltpu.sync_copy(table_hbm.at[:, pl.ds(col0, _HALF)], tab_sp)
    c_idx = pltpu.async_copy(idx_hbm.at[pl.ds(row_base, _B_PER_W)],
                             idx_v.at[pl.ds(0, _B_PER_W)], sem1)
    plsc.subcore_barrier()
    pltpu.sync_copy(tab_sp, tab_v)
    c_idx.wait()

    iota16 = lax.iota(jnp.int32, 16)
    zeros16 = jnp.zeros((16,), jnp.int32)

    @pl.loop(0, _NBLK)
    def _blocks(blk):
        b = blk & 1
        r0 = blk * _ROWS
        ta0 = g * (_B_PER_W // 8) + blk * (_ROWS // 8)

        @pl.when(blk >= 2)
        def _wait():
            pltpu.make_async_copy(
                buf_v.at[b],
                out_hbm.at[pl.ds(ta0, _ROWS // 8), pl.ds(tb0, _TBH)],
                wsem.at[b]).wait()

        @plsc.parallel_loop(0, _ROWS)
        def _row(r):
            idxv = idx_v[pl.ds(r0 + r, 16)]
            bcast = idxv.at[zeros16].get(mode="promise_in_bounds")
            ra = r // 8
            rs = r % 8
            for k in range(_HALF // 16):
                v = plsc.load_gather(tab_v, [bcast, iota16 + k * 16])
                dst = buf_v.at[b, ra, k // 8, rs]
                dst[pl.ds((k % 8) * 16, 16)] = v

        pltpu.async_copy(
            buf_v.at[b],
            out_hbm.at[pl.ds(ta0, _ROWS // 8), pl.ds(tb0, _TBH)],
            wsem.at[b])

    for b in range(2):
        pltpu.make_async_copy(
            buf_v.at[b],
            out_hbm.at[pl.ds(0, _ROWS // 8), pl.ds(tb0, _TBH)],
            wsem.at[b]).wait()


def kernel(anchor_embeddings, indices):
    tiled = _sc_lookup(anchor_embeddings, indices.astype(jnp.int32))
    return tiled.transpose(0, 2, 1, 3).reshape(_BATCH, _EMBED_DIM)
